# Initial kernel scaffold; baseline (speedup 1.0000x reference)
import jax
import jax.numpy as jnp
from jax.experimental import pallas as pl


def kernel(heads, rels, negative_sample, subsampling_weight, center_embedding, offset_embedding, center_mul, center_add, offset_mul, offset_add):
    raise NotImplementedError("write your pallas kernel here")


# speedup vs baseline: 10.1576x; 10.1576x over previous
import dataclasses
import functools

import jax
import jax.numpy as jnp
import numpy as np
from jax import lax
from jax.experimental import pallas as pl
from jax.experimental.pallas import tpu as pltpu
from jax.experimental.pallas import tpu_sc as plsc

_B = 4096
_NNEG = 1024
_DIM = 64
_GAMMA = 24.0

_NC = 2
_NS = 16
_L = 16
_NW = _NC * _NS
_BPW = _B // _NW
_W = 128
_WPB = _NNEG // _W

def _body(heads_hbm, rels_hbm, neg_hbm, cen_hbm, off_hbm,
          cm_hbm, ca_hbm, om_hbm, oa_hbm, out_hbm,
          hidx_v, ridx_v, hc_v, ho_v, cmv_v, cav_v, omv_v, oav_v,
          q_v, idx_v, rows_v, out_v, gsem):
    wid = lax.axis_index("s") * _NC + lax.axis_index("c")
    base_b = wid * _BPW

    pltpu.sync_copy(heads_hbm.at[pl.ds(base_b, _BPW)], hidx_v)
    pltpu.sync_copy(rels_hbm.at[pl.ds(base_b, _BPW)], ridx_v)
    cps = [
        pltpu.async_copy(cen_hbm.at[hidx_v], hc_v, gsem),
        pltpu.async_copy(off_hbm.at[hidx_v], ho_v, gsem),
        pltpu.async_copy(cm_hbm.at[ridx_v], cmv_v, gsem),
        pltpu.async_copy(ca_hbm.at[ridx_v], cav_v, gsem),
        pltpu.async_copy(om_hbm.at[ridx_v], omv_v, gsem),
        pltpu.async_copy(oa_hbm.at[ridx_v], oav_v, gsem),
    ]
    for cp in cps:
        cp.wait()

    @pl.loop(0, _BPW)
    def _(i):
        for k in range(_DIM // _L):
            sl = pl.ds(k * _L, _L)
            qc = hc_v[i, sl] * cmv_v[i, sl] + cav_v[i, sl]
            qo = jnp.maximum(ho_v[i, sl] * omv_v[i, sl] + oav_v[i, sl], 0.0)
            q_v[i, sl] = qc
            q_v[i, pl.ds(_DIM + k * _L, _L)] = 0.5 * qo

    @pl.loop(0, _BPW)
    def _(bl):
        b = base_b + bl
        pltpu.sync_copy(neg_hbm.at[pl.ds(b * _WPB, _WPB)], idx_v)
        qcs = [q_v[bl, pl.ds(k * _L, _L)] for k in range(_DIM // _L)]
        qos = [q_v[bl, pl.ds(_DIM + k * _L, _L)] for k in range(_DIM // _L)]
        lane = lax.iota(jnp.int32, _L)
        for w in range(_WPB):
            pltpu.async_copy(cen_hbm.at[idx_v.at[w]], rows_v, gsem).wait()

            @pl.loop(0, _W // _L)
            def _(g):
                ob = jnp.zeros((_L,), jnp.float32)
                for rr in range(_L):
                    r = g * _L + rr
                    acc = None
                    for k in range(_DIM // _L):
                        x = rows_v[r, pl.ds(k * _L, _L)]
                        d = jnp.abs(x - qcs[k])
                        v = jnp.maximum(d - qos[k], 0.5 * d)
                        acc = v if acc is None else acc + v
                    tot = jnp.sum(acc)
                    ob = jnp.where(lane == rr, tot, ob)
                out_v[pl.ds(w * _W + g * _L, _L)] = _GAMMA - ob

        pltpu.sync_copy(out_v, out_hbm.at[pl.ds(b * _NNEG, _NNEG)])


@jax.jit
def _sc_call(heads, rels, neg, cen, off, cm, ca, om, oa):
    mesh = plsc.VectorSubcoreMesh(core_axis_name="c", subcore_axis_name="s")
    cp = pltpu.CompilerParams()
    if "needs_layout_passes" in pltpu.CompilerParams.__dataclass_fields__:
        cp = dataclasses.replace(cp, needs_layout_passes=False)
    if "use_tc_tiling_on_sc" in pltpu.CompilerParams.__dataclass_fields__:
        cp = dataclasses.replace(cp, use_tc_tiling_on_sc=False)
    f = pl.kernel(
        _body,
        out_type=jax.ShapeDtypeStruct((_B * _NNEG,), jnp.float32),
        mesh=mesh,
        scratch_types=[
            pltpu.VMEM((_BPW,), jnp.int32),
            pltpu.VMEM((_BPW,), jnp.int32),
            pltpu.VMEM((_BPW, _DIM), jnp.float32),
            pltpu.VMEM((_BPW, _DIM), jnp.float32),
            pltpu.VMEM((_BPW, _DIM), jnp.float32),
            pltpu.VMEM((_BPW, _DIM), jnp.float32),
            pltpu.VMEM((_BPW, _DIM), jnp.float32),
            pltpu.VMEM((_BPW, _DIM), jnp.float32),
            pltpu.VMEM((_BPW, 2 * _DIM), jnp.float32),
            pltpu.VMEM((_WPB, _W), jnp.int32),
            pltpu.VMEM((_W, _DIM), jnp.float32),
            pltpu.VMEM((_NNEG,), jnp.float32),
            pltpu.SemaphoreType.DMA,
        ],
        compiler_params=cp,
    )
    return f(heads, rels, neg, cen, off, cm, ca, om, oa)


def kernel(heads, rels, negative_sample, subsampling_weight,
           center_embedding, offset_embedding,
           center_mul, center_add, offset_mul, offset_add):
    del subsampling_weight
    heads = heads.astype(jnp.int32)
    rels = rels.astype(jnp.int32)
    neg = negative_sample.astype(jnp.int32).reshape(_B * _WPB, _W)
    out = _sc_call(heads, rels, neg, center_embedding, offset_embedding,
                   center_mul, center_add, offset_mul, offset_add)
    return out.reshape(_B, _NNEG)


# speedup vs baseline: 15.0698x; 1.4836x over previous
import dataclasses
import functools

import jax
import jax.numpy as jnp
import numpy as np
from jax import lax
from jax.experimental import pallas as pl
from jax.experimental.pallas import tpu as pltpu
from jax.experimental.pallas import tpu_sc as plsc

_B = 4096
_NNEG = 1024
_DIM = 64
_GAMMA = 24.0

_NC = 2
_NS = 16
_L = 16
_NW = _NC * _NS
_BPW = _B // _NW
_W = 128
_WPB = _NNEG // _W

def _body(heads_hbm, rels_hbm, neg_hbm, cen_hbm, off_hbm,
          cm_hbm, ca_hbm, om_hbm, oa_hbm, out_hbm,
          hidx_v, ridx_v, hc_v, ho_v, cmv_v, cav_v, omv_v, oav_v,
          q_v, idx_v, rows_v, out_v, gsem, gsem0, gsem1):
    wid = lax.axis_index("s") * _NC + lax.axis_index("c")
    base_b = wid * _BPW

    pltpu.sync_copy(heads_hbm.at[pl.ds(base_b, _BPW)], hidx_v)
    pltpu.sync_copy(rels_hbm.at[pl.ds(base_b, _BPW)], ridx_v)
    cps = [
        pltpu.async_copy(cen_hbm.at[hidx_v], hc_v, gsem),
        pltpu.async_copy(off_hbm.at[hidx_v], ho_v, gsem),
        pltpu.async_copy(cm_hbm.at[ridx_v], cmv_v, gsem),
        pltpu.async_copy(ca_hbm.at[ridx_v], cav_v, gsem),
        pltpu.async_copy(om_hbm.at[ridx_v], omv_v, gsem),
        pltpu.async_copy(oa_hbm.at[ridx_v], oav_v, gsem),
    ]
    for cp in cps:
        cp.wait()

    @pl.loop(0, _BPW)
    def _(i):
        for k in range(_DIM // _L):
            sl = pl.ds(k * _L, _L)
            qc = hc_v[i, sl] * cmv_v[i, sl] + cav_v[i, sl]
            qo = jnp.maximum(ho_v[i, sl] * omv_v[i, sl] + oav_v[i, sl], 0.0)
            q_v[i, sl] = qc
            q_v[i, pl.ds(_DIM + k * _L, _L)] = 0.5 * qo

    @pl.loop(0, _BPW)
    def _(bl):
        b = base_b + bl
        pltpu.sync_copy(neg_hbm.at[pl.ds(b * _WPB, _WPB)], idx_v)
        qcs = [q_v[bl, pl.ds(k * _L, _L)] for k in range(_DIM // _L)]
        qos = [q_v[bl, pl.ds(_DIM + k * _L, _L)] for k in range(_DIM // _L)]
        lane = lax.iota(jnp.int32, _L)
        gsems = [gsem0, gsem1]

        def gather(w):
            return pltpu.make_async_copy(
                cen_hbm.at[idx_v.at[w]], rows_v.at[w % 2], gsems[w % 2])

        gather(0).start()
        for w in range(_WPB):
            if w + 1 < _WPB:
                gather(w + 1).start()
            gather(w).wait()

            @pl.loop(0, _W // _L)
            def _(g):
                ob = jnp.zeros((_L,), jnp.float32)
                for rr in range(_L):
                    r = g * _L + rr
                    acc = None
                    for k in range(_DIM // _L):
                        x = rows_v[w % 2, r, pl.ds(k * _L, _L)]
                        d = jnp.abs(x - qcs[k])
                        v = jnp.maximum(d - qos[k], 0.5 * d)
                        acc = v if acc is None else acc + v
                    tot = jnp.sum(acc)
                    ob = jnp.where(lane == rr, tot, ob)
                out_v[pl.ds(w * _W + g * _L, _L)] = _GAMMA - ob

        pltpu.sync_copy(out_v, out_hbm.at[pl.ds(b * _NNEG, _NNEG)])


@jax.jit
def _sc_call(heads, rels, neg, cen, off, cm, ca, om, oa):
    mesh = plsc.VectorSubcoreMesh(core_axis_name="c", subcore_axis_name="s")
    cp = pltpu.CompilerParams()
    if "needs_layout_passes" in pltpu.CompilerParams.__dataclass_fields__:
        cp = dataclasses.replace(cp, needs_layout_passes=False)
    if "use_tc_tiling_on_sc" in pltpu.CompilerParams.__dataclass_fields__:
        cp = dataclasses.replace(cp, use_tc_tiling_on_sc=False)
    f = pl.kernel(
        _body,
        out_type=jax.ShapeDtypeStruct((_B * _NNEG,), jnp.float32),
        mesh=mesh,
        scratch_types=[
            pltpu.VMEM((_BPW,), jnp.int32),
            pltpu.VMEM((_BPW,), jnp.int32),
            pltpu.VMEM((_BPW, _DIM), jnp.float32),
            pltpu.VMEM((_BPW, _DIM), jnp.float32),
            pltpu.VMEM((_BPW, _DIM), jnp.float32),
            pltpu.VMEM((_BPW, _DIM), jnp.float32),
            pltpu.VMEM((_BPW, _DIM), jnp.float32),
            pltpu.VMEM((_BPW, _DIM), jnp.float32),
            pltpu.VMEM((_BPW, 2 * _DIM), jnp.float32),
            pltpu.VMEM((_WPB, _W), jnp.int32),
            pltpu.VMEM((2, _W, _DIM), jnp.float32),
            pltpu.VMEM((_NNEG,), jnp.float32),
            pltpu.SemaphoreType.DMA,
            pltpu.SemaphoreType.DMA,
            pltpu.SemaphoreType.DMA,
        ],
        compiler_params=cp,
    )
    return f(heads, rels, neg, cen, off, cm, ca, om, oa)


def kernel(heads, rels, negative_sample, subsampling_weight,
           center_embedding, offset_embedding,
           center_mul, center_add, offset_mul, offset_add):
    del subsampling_weight
    heads = heads.astype(jnp.int32)
    rels = rels.astype(jnp.int32)
    neg = negative_sample.astype(jnp.int32).reshape(_B * _WPB, _W)
    out = _sc_call(heads, rels, neg, center_embedding, offset_embedding,
                   center_mul, center_add, offset_mul, offset_add)
    return out.reshape(_B, _NNEG)


# speedup vs baseline: 19.7598x; 1.3112x over previous
import dataclasses

import jax
import jax.numpy as jnp
from jax import lax
from jax.experimental import pallas as pl
from jax.experimental.pallas import tpu as pltpu
from jax.experimental.pallas import tpu_sc as plsc

_B = 4096
_NNEG = 1024
_DIM = 64
_GAMMA = 24.0

_NC = 2
_NS = 16
_L = 16
_NW = _NC * _NS
_BPW = _B // _NW
_W = 128
_WPB = _NNEG // _W
_LB = 2 * _L


def _compiler_params():
    cp = pltpu.CompilerParams()
    fields = pltpu.CompilerParams.__dataclass_fields__
    if "needs_layout_passes" in fields:
        cp = dataclasses.replace(cp, needs_layout_passes=False)
    if "use_tc_tiling_on_sc" in fields:
        cp = dataclasses.replace(cp, use_tc_tiling_on_sc=False)
    return cp


def _q_body(heads_hbm, rels_hbm, cen_hbm, off_hbm,
            cm_hbm, ca_hbm, om_hbm, oa_hbm, q_hbm,
            hidx_v, ridx_v, hc_v, ho_v, cmv_v, cav_v, omv_v, oav_v,
            q_v, gsem):
    wid = lax.axis_index("s") * _NC + lax.axis_index("c")
    base_b = wid * _BPW

    pltpu.sync_copy(heads_hbm.at[pl.ds(base_b, _BPW)], hidx_v)
    pltpu.sync_copy(rels_hbm.at[pl.ds(base_b, _BPW)], ridx_v)
    cps = [
        pltpu.async_copy(cen_hbm.at[hidx_v], hc_v, gsem),
        pltpu.async_copy(off_hbm.at[hidx_v], ho_v, gsem),
        pltpu.async_copy(cm_hbm.at[ridx_v], cmv_v, gsem),
        pltpu.async_copy(ca_hbm.at[ridx_v], cav_v, gsem),
        pltpu.async_copy(om_hbm.at[ridx_v], omv_v, gsem),
        pltpu.async_copy(oa_hbm.at[ridx_v], oav_v, gsem),
    ]
    for cp in cps:
        cp.wait()

    @pl.loop(0, _BPW)
    def _(i):
        for k in range(_DIM // _L):
            sl = pl.ds(k * _L, _L)
            qc = hc_v[i, sl] * cmv_v[i, sl] + cav_v[i, sl]
            qo = jnp.maximum(ho_v[i, sl] * omv_v[i, sl] + oav_v[i, sl], 0.0)
            q_v[i, sl] = qc
            q_v[i, pl.ds(_DIM + k * _L, _L)] = 0.5 * qo

    pltpu.sync_copy(q_v, q_hbm.at[pl.ds(base_b, _BPW)])


def _main_body(q_hbm, neg_hbm, cen_hbm, out_hbm,
               q_v, idx_v, rows_v, out_v,
               gsem0, gsem1, isem0, isem1, osem0, osem1):
    wid = lax.axis_index("s") * _NC + lax.axis_index("c")
    base_b = wid * _BPW
    gsems = [gsem0, gsem1]
    isems = [isem0, isem1]
    osems = [osem0, osem1]
    lane = lax.iota(jnp.int32, _L)
    half = jnp.bfloat16(0.5)

    pltpu.sync_copy(q_hbm.at[pl.ds(base_b, _BPW)], q_v)

    def idx_copy(b, slot):
        return pltpu.make_async_copy(
            neg_hbm.at[pl.ds(b * _WPB, _WPB)], idx_v.at[slot], isems[slot])

    def out_copy(b, slot):
        return pltpu.make_async_copy(
            out_v.at[slot], out_hbm.at[pl.ds(b * _NNEG, _NNEG)], osems[slot])

    idx_copy(base_b, 0).start()

    @pl.loop(0, _BPW, step=2)
    def _(bl):
        for h in range(2):
            bb = bl + h
            b = base_b + bb
            idx_copy(b, h).wait()

            @pl.when(bb + 1 < _BPW)
            def _():
                idx_copy(b + 1, (h + 1) % 2).start()

            @pl.when(bb >= 2)
            def _():
                out_copy(b - 2, h).wait()

            qcs = [q_v[bb, pl.ds(k * _LB, _LB)] for k in range(2)]
            qos = [q_v[bb, pl.ds(_DIM + k * _LB, _LB)] for k in range(2)]

            def gather(w):
                return pltpu.make_async_copy(
                    cen_hbm.at[idx_v.at[h, w]], rows_v.at[w % 2],
                    gsems[w % 2])

            gather(0).start()
            for w in range(_WPB):
                if w + 1 < _WPB:
                    gather(w + 1).start()
                gather(w).wait()

                @pl.loop(0, _W // _L)
                def _(g):
                    ob = jnp.zeros((_L,), jnp.float32)
                    for rr in range(_L):
                        r = g * _L + rr
                        acc_bf = None
                        for k in range(2):
                            x = rows_v[w % 2, r, pl.ds(k * _LB, _LB)]
                            d = jnp.abs(x - qcs[k])
                            v = jnp.maximum(d - qos[k], half * d)
                            acc_bf = v if acc_bf is None else acc_bf + v
                        pa, pb = plsc.unpack(
                            acc_bf, format=plsc.PackFormat.INTERLEAVED)
                        tot = jnp.sum(pa + pb)
                        ob = jnp.where(lane == rr, tot, ob)
                    out_v[h, pl.ds(w * _W + g * _L, _L)] = _GAMMA - ob

            out_copy(b, h).start()

    out_copy(base_b, 0).wait()
    out_copy(base_b, 1).wait()


@jax.jit
def _sc_call(heads, rels, neg, cen, cen_bf, off, cm, ca, om, oa):
    mesh = plsc.VectorSubcoreMesh(core_axis_name="c", subcore_axis_name="s")
    cp = _compiler_params()

    q = pl.kernel(
        _q_body,
        out_type=jax.ShapeDtypeStruct((_B, 2 * _DIM), jnp.float32),
        mesh=mesh,
        scratch_types=[
            pltpu.VMEM((_BPW,), jnp.int32),
            pltpu.VMEM((_BPW,), jnp.int32),
            pltpu.VMEM((_BPW, _DIM), jnp.float32),
            pltpu.VMEM((_BPW, _DIM), jnp.float32),
            pltpu.VMEM((_BPW, _DIM), jnp.float32),
            pltpu.VMEM((_BPW, _DIM), jnp.float32),
            pltpu.VMEM((_BPW, _DIM), jnp.float32),
            pltpu.VMEM((_BPW, _DIM), jnp.float32),
            pltpu.VMEM((_BPW, 2 * _DIM), jnp.float32),
            pltpu.SemaphoreType.DMA,
        ],
        compiler_params=cp,
    )(heads, rels, cen, off, cm, ca, om, oa)

    q_bf = q.astype(jnp.bfloat16)

    out = pl.kernel(
        _main_body,
        out_type=jax.ShapeDtypeStruct((_B * _NNEG,), jnp.float32),
        mesh=mesh,
        scratch_types=[
            pltpu.VMEM((_BPW, 2 * _DIM), jnp.bfloat16),
            pltpu.VMEM((2, _WPB, _W), jnp.int32),
            pltpu.VMEM((2, _W, _DIM), jnp.bfloat16),
            pltpu.VMEM((2, _NNEG), jnp.float32),
            pltpu.SemaphoreType.DMA,
            pltpu.SemaphoreType.DMA,
            pltpu.SemaphoreType.DMA,
            pltpu.SemaphoreType.DMA,
            pltpu.SemaphoreType.DMA,
            pltpu.SemaphoreType.DMA,
        ],
        compiler_params=cp,
    )(q_bf, neg, cen_bf)

    return out


def kernel(heads, rels, negative_sample, subsampling_weight,
           center_embedding, offset_embedding,
           center_mul, center_add, offset_mul, offset_add):
    del subsampling_weight
    heads = heads.astype(jnp.int32)
    rels = rels.astype(jnp.int32)
    neg = negative_sample.astype(jnp.int32).reshape(_B * _WPB, _W)
    cen_bf = center_embedding.astype(jnp.bfloat16)
    out = _sc_call(heads, rels, neg, center_embedding, cen_bf,
                   offset_embedding, center_mul, center_add,
                   offset_mul, offset_add)
    return out.reshape(_B, _NNEG)
